# baseline (speedup 1.0000x reference)
import jax
import jax.numpy as jnp
from jax import lax
from jax.experimental import pallas as pl
from jax.experimental.pallas import tpu as pltpu
from jax.experimental.pallas import tpu_sc as plsc

_BATCH = 16384
_EMBED_DIM = 64
_VOCAB = 100001
_NUM_CORES = 2
_NUM_SUBCORES = 16
_NUM_WORKERS = _NUM_CORES * _NUM_SUBCORES
_DIMS_PER_W = _EMBED_DIM // _NUM_WORKERS
_CHUNK = 4096
_NUM_CHUNKS = _BATCH // _CHUNK
_LANES = 16
_UNROLL = 16
_TAIL_START = 75264 + 24704
_ROW_SPLITS = (0, 25088, 50176, 75264, _TAIL_START)
_IDX_SPLITS = (0, 8192, _BATCH)
_ROW_PAD = _TAIL_START + 128


def _stage_row(tblt_hbm, tail_hbm, row_v, d, sem):
    dmas = [
        pltpu.async_copy(
            tblt_hbm.at[d, pl.ds(lo, hi - lo)],
            row_v.at[pl.ds(lo, hi - lo)],
            sem,
        )
        for lo, hi in zip(_ROW_SPLITS[:-1], _ROW_SPLITS[1:])
    ]
    dmas.append(
        pltpu.async_copy(
            tail_hbm.at[d], row_v.at[pl.ds(_TAIL_START, 128)], sem
        )
    )
    return dmas


def _gather_body(
    idx_hbm, tblt_hbm, tail_hbm, outt_hbm, row_v, idx_v, out_v, rsem, isem, osem
):
    wid = lax.axis_index("s") * _NUM_CORES + lax.axis_index("c")
    row_dmas = _stage_row(tblt_hbm, tail_hbm, row_v, wid * _DIMS_PER_W, rsem)
    idx_dmas = [
        pltpu.async_copy(
            idx_hbm.at[pl.ds(lo, hi - lo)], idx_v.at[pl.ds(lo, hi - lo)], isem
        )
        for lo, hi in zip(_IDX_SPLITS[:-1], _IDX_SPLITS[1:])
    ]
    for dma in idx_dmas:
        dma.wait()
    pending = []
    for j in range(_DIMS_PER_W):
        d = wid * _DIMS_PER_W + j
        for dma in row_dmas:
            dma.wait()
        for c in range(_NUM_CHUNKS):
            buf = (j * _NUM_CHUNKS + c) % 2
            if len(pending) >= 2:
                pending.pop(0).wait()

            @plsc.parallel_loop(0, _CHUNK, step=_LANES, unroll=_UNROLL)
            def _(o, _c=c, _buf=buf):
                iv = idx_v[pl.ds(_c * _CHUNK + o, _LANES)]
                out_v[_buf, pl.ds(o, _LANES)] = plsc.load_gather(row_v, [iv])

            pending.append(
                pltpu.async_copy(
                    out_v.at[buf],
                    outt_hbm.at[d, pl.ds(c * _CHUNK, _CHUNK)],
                    osem,
                )
            )
        if j + 1 < _DIMS_PER_W:
            row_dmas = _stage_row(tblt_hbm, tail_hbm, row_v, d + 1, rsem)
    for p in pending:
        p.wait()


@jax.jit
def _gather(indices, table):
    mesh = plsc.VectorSubcoreMesh(
        core_axis_name="c",
        subcore_axis_name="s",
        num_cores=_NUM_CORES,
        num_subcores=_NUM_SUBCORES,
    )
    table_t = table.T
    tail = jnp.pad(
        lax.slice(table_t, (0, _TAIL_START), (_EMBED_DIM, _VOCAB)),
        ((0, 0), (0, 128 - (_VOCAB - _TAIL_START))),
    )
    out_t = pl.kernel(
        _gather_body,
        out_type=jax.ShapeDtypeStruct((_EMBED_DIM, _BATCH), jnp.float32),
        mesh=mesh,
        scratch_types=[
            pltpu.VMEM((_ROW_PAD,), jnp.float32),
            pltpu.VMEM((_BATCH,), jnp.int32),
            pltpu.VMEM((2, _CHUNK), jnp.float32),
            pltpu.SemaphoreType.DMA,
            pltpu.SemaphoreType.DMA,
            pltpu.SemaphoreType.DMA,
        ],
        compiler_params=pltpu.CompilerParams(needs_layout_passes=False),
    )(indices, table_t, tail)
    return out_t.T


def kernel(indices, table):
    return _gather(indices.astype(jnp.int32), table)


# speedup vs baseline: 1.1036x; 1.1036x over previous
import jax
import jax.numpy as jnp
from jax import lax
from jax.experimental import pallas as pl
from jax.experimental.pallas import tpu as pltpu
from jax.experimental.pallas import tpu_sc as plsc

_BATCH = 16384
_EMBED_DIM = 64
_VOCAB = 100001
_NUM_CORES = 2
_NUM_SUBCORES = 16
_NUM_WORKERS = _NUM_CORES * _NUM_SUBCORES
_DIMS_PER_W = _EMBED_DIM // _NUM_WORKERS
_CHUNK = 4096
_NUM_CHUNKS = _BATCH // _CHUNK
_LANES = 16
_UNROLL = 8


def _gather_body(idx_hbm, tblt_hbm, outt_hbm, row_v, idx_v, out_v, idx_sp, rsem, osem):
    sid = lax.axis_index("s")
    wid = sid * _NUM_CORES + lax.axis_index("c")
    with jax.named_scope("phase_idx_stage"):
        row_dma = pltpu.async_copy(tblt_hbm.at[wid * _DIMS_PER_W], row_v, rsem)

        @pl.when(sid == 0)
        def _():
            pltpu.sync_copy(idx_hbm, idx_sp)

        plsc.subcore_barrier()
        pltpu.sync_copy(idx_sp, idx_v)
    pending = []
    for j in range(_DIMS_PER_W):
        d = wid * _DIMS_PER_W + j
        with jax.named_scope("phase_row_wait"):
            row_dma.wait()
        for c in range(_NUM_CHUNKS):
            buf = (j * _NUM_CHUNKS + c) % 2
            if len(pending) >= 2:
                pending.pop(0).wait()

            with jax.named_scope("phase_gather"):

                @plsc.parallel_loop(0, _CHUNK, step=_LANES, unroll=_UNROLL)
                def _(o, _c=c, _buf=buf):
                    iv = idx_v[pl.ds(_c * _CHUNK + o, _LANES)]
                    out_v[_buf, pl.ds(o, _LANES)] = plsc.load_gather(
                        row_v, [iv]
                    )
            pending.append(
                pltpu.async_copy(
                    out_v.at[buf],
                    outt_hbm.at[d, pl.ds(c * _CHUNK, _CHUNK)],
                    osem,
                )
            )
        if j + 1 < _DIMS_PER_W:
            row_dma = pltpu.async_copy(tblt_hbm.at[d + 1], row_v, rsem)
    for p in pending:
        p.wait()


@jax.jit
def _gather(indices, table):
    mesh = plsc.VectorSubcoreMesh(
        core_axis_name="c",
        subcore_axis_name="s",
        num_cores=_NUM_CORES,
        num_subcores=_NUM_SUBCORES,
    )
    out_t = pl.kernel(
        _gather_body,
        out_type=jax.ShapeDtypeStruct((_EMBED_DIM, _BATCH), jnp.float32),
        mesh=mesh,
        scratch_types=[
            pltpu.VMEM((_VOCAB,), jnp.float32),
            pltpu.VMEM((_BATCH,), jnp.int32),
            pltpu.VMEM((2, _CHUNK), jnp.float32),
            pltpu.VMEM_SHARED((_BATCH,), jnp.int32),
            pltpu.SemaphoreType.DMA,
            pltpu.SemaphoreType.DMA,
        ],
        compiler_params=pltpu.CompilerParams(needs_layout_passes=False),
    )(indices, table.T)
    return out_t.T


def kernel(indices, table):
    return _gather(indices.astype(jnp.int32), table)
